# Initial kernel scaffold; baseline (speedup 1.0000x reference)
import jax
import jax.numpy as jnp
from jax.experimental import pallas as pl


def kernel(token_ids, token_matrix, segment_matrix, pos_matrix):
    raise NotImplementedError("write your pallas kernel here")


# speedup vs baseline: 1.4446x; 1.4446x over previous
import functools

import jax
import jax.numpy as jnp
from jax import lax
from jax.experimental import pallas as pl
from jax.experimental.pallas import tpu as pltpu
from jax.experimental.pallas import tpu_sc as plsc

SEP = 103
D = 768
SEQ = 512
B = 32
L = 16
NC, NS = 2, 16
CS = 32
NCHUNK = SEQ // CS
DCH = D // L


def _body(ids_hbm, table_hbm, seg_hbm, pos_hbm, out_hbm,
          idx_v, t_v, seg_v, tok_buf, pos_buf, perm_v, sem_g, sem_p):
    c = lax.axis_index("c")
    s = lax.axis_index("s")
    w = s * NC + c

    pltpu.sync_copy(ids_hbm.at[w], idx_v)
    pltpu.sync_copy(seg_hbm, seg_v)

    iota = lax.iota(jnp.int32, L)

    def seg_body(i, carry):
        base = i * L
        tok = idx_v[pl.ds(base, L)]
        p = jnp.where(tok == SEP, 1, 0).astype(jnp.int32)
        for sh in (1, 2, 4, 8):
            perm_v[...] = p
            p = p | plsc.load_gather(perm_v, [jnp.maximum(iota - sh, 0)])
        perm_v[...] = p
        excl = plsc.load_gather(perm_v, [jnp.maximum(iota - 1, 0)])
        excl = jnp.where(iota == 0, 0, excl)
        t_v[pl.ds(base, L)] = (carry | excl).astype(jnp.float32)
        last = plsc.load_gather(perm_v, [jnp.full((L,), L - 1, jnp.int32)])
        return carry | last

    lax.fori_loop(0, SEQ // L, seg_body, jnp.zeros((L,), jnp.int32))

    def chunk(cidx, _):
        base = cidx * CS
        cp_g = pltpu.async_copy(table_hbm.at[idx_v.at[pl.ds(base, CS)]],
                                tok_buf, sem_g)
        cp_p = pltpu.async_copy(pos_hbm.at[pl.ds(base, CS)], pos_buf, sem_p)
        cp_g.wait()
        cp_p.wait()

        def dloop(j, _):
            dsl = pl.ds(j * L, L)
            seg0 = seg_v[0, dsl]
            dseg = seg_v[1, dsl] - seg0
            def gloop(g, _):
                tvec = t_v[pl.ds(base + g * L, L)]
                for k in range(L):
                    si = g * L + k
                    r = (tok_buf[si, dsl] + pos_buf[si, dsl]
                         + (seg0 + tvec[k] * dseg))
                    tok_buf[si, dsl] = r
                return 0
            return lax.fori_loop(0, CS // L, gloop, 0)

        lax.fori_loop(0, DCH, dloop, 0)
        pltpu.sync_copy(tok_buf, out_hbm.at[w, pl.ds(base, CS)])
        return 0

    lax.fori_loop(0, NCHUNK, chunk, 0)


@jax.jit
def kernel(token_ids, token_matrix, segment_matrix, pos_matrix):
    mesh = plsc.VectorSubcoreMesh(core_axis_name="c", subcore_axis_name="s",
                                  num_cores=NC, num_subcores=NS)
    run = pl.kernel(
        _body,
        out_type=jax.ShapeDtypeStruct((B, SEQ, D), jnp.float32),
        mesh=mesh,
        scratch_types=[
            pltpu.VMEM((SEQ,), jnp.int32),
            pltpu.VMEM((SEQ,), jnp.float32),
            pltpu.VMEM((2, D), jnp.float32),
            pltpu.VMEM((CS, D), jnp.float32),
            pltpu.VMEM((CS, D), jnp.float32),
            pltpu.VMEM((L,), jnp.int32),
            pltpu.SemaphoreType.DMA,
            pltpu.SemaphoreType.DMA,
        ],
        compiler_params=pltpu.CompilerParams(needs_layout_passes=False),
    )
    return run(token_ids.astype(jnp.int32), token_matrix, segment_matrix,
               pos_matrix)


# speedup vs baseline: 1.8946x; 1.3116x over previous
import jax
import jax.numpy as jnp
from jax import lax
from jax.experimental import pallas as pl
from jax.experimental.pallas import tpu as pltpu
from jax.experimental.pallas import tpu_sc as plsc

SEP = 103
D = 768
SEQ = 512
B = 32
L = 16
NC, NS = 2, 16
CS = 32
NCHUNK = SEQ // CS
DCH = D // L


def _body(ids_hbm, table_hbm, seg_hbm, pos_hbm, out_hbm,
          idx_v, t_v, seg_v, tok0, tok1, pos0, pos1, perm_v,
          sg0, sg1, sp0, sp1, so0, so1):
    c = lax.axis_index("c")
    s = lax.axis_index("s")
    w = s * NC + c

    pltpu.sync_copy(ids_hbm.at[w], idx_v)
    pltpu.sync_copy(seg_hbm, seg_v)

    iota = lax.iota(jnp.int32, L)

    def seg_body(i, carry):
        base = i * L
        tok = idx_v[pl.ds(base, L)]
        p = jnp.where(tok == SEP, 1, 0).astype(jnp.int32)
        for sh in (1, 2, 4, 8):
            perm_v[...] = p
            p = p | plsc.load_gather(perm_v, [jnp.maximum(iota - sh, 0)])
        perm_v[...] = p
        excl = plsc.load_gather(perm_v, [jnp.maximum(iota - 1, 0)])
        excl = jnp.where(iota == 0, 0, excl)
        t_v[pl.ds(base, L)] = (carry | excl).astype(jnp.float32)
        last = plsc.load_gather(perm_v, [jnp.full((L,), L - 1, jnp.int32)])
        return carry | last

    lax.fori_loop(0, SEQ // L, seg_body, jnp.zeros((L,), jnp.int32))

    def prefetch(cidx, tok_b, pos_b, sem_g, sem_p):
        base = cidx * CS
        pltpu.async_copy(table_hbm.at[idx_v.at[pl.ds(base, CS)]],
                         tok_b, sem_g)
        pltpu.async_copy(pos_hbm.at[pl.ds(base, CS)], pos_b, sem_p)

    def compute(cidx, tok_b, pos_b, sem_g, sem_p, sem_o):
        base = cidx * CS
        pltpu.make_async_copy(pos_hbm.at[pl.ds(base, CS)], pos_b,
                              sem_p).wait()
        pltpu.make_async_copy(table_hbm.at[idx_v.at[pl.ds(base, CS)]],
                              tok_b, sem_g).wait()

        def dloop(j, _):
            dsl = pl.ds(j * L, L)
            seg0 = seg_v[0, dsl]
            dseg = seg_v[1, dsl] - seg0
            def gloop(g, _):
                tvec = t_v[pl.ds(base + g * L, L)]
                for k in range(L):
                    si = g * L + k
                    r = (tok_b[si, dsl] + pos_b[si, dsl]
                         + (seg0 + tvec[k] * dseg))
                    tok_b[si, dsl] = r
                return 0
            return lax.fori_loop(0, CS // L, gloop, 0)

        lax.fori_loop(0, DCH, dloop, 0)
        pltpu.async_copy(tok_b, out_hbm.at[w, pl.ds(base, CS)], sem_o)

    def wait_out(tok_b, sem_o):
        pltpu.make_async_copy(tok_b, out_hbm.at[w, pl.ds(0, CS)],
                              sem_o).wait()

    prefetch(0, tok0, pos0, sg0, sp0)
    prefetch(1, tok1, pos1, sg1, sp1)

    def pair(h, _):
        compute(2 * h, tok0, pos0, sg0, sp0, so0)

        @pl.when(h < NCHUNK // 2 - 1)
        def _():
            wait_out(tok0, so0)
            prefetch(2 * h + 2, tok0, pos0, sg0, sp0)

        compute(2 * h + 1, tok1, pos1, sg1, sp1, so1)

        @pl.when(h < NCHUNK // 2 - 1)
        def _():
            wait_out(tok1, so1)
            prefetch(2 * h + 3, tok1, pos1, sg1, sp1)

        return 0

    lax.fori_loop(0, NCHUNK // 2, pair, 0)
    wait_out(tok0, so0)
    wait_out(tok1, so1)


@jax.jit
def kernel(token_ids, token_matrix, segment_matrix, pos_matrix):
    mesh = plsc.VectorSubcoreMesh(core_axis_name="c", subcore_axis_name="s",
                                  num_cores=NC, num_subcores=NS)
    run = pl.kernel(
        _body,
        out_type=jax.ShapeDtypeStruct((B, SEQ, D), jnp.float32),
        mesh=mesh,
        scratch_types=[
            pltpu.VMEM((SEQ,), jnp.int32),
            pltpu.VMEM((SEQ,), jnp.float32),
            pltpu.VMEM((2, D), jnp.float32),
            pltpu.VMEM((CS, D), jnp.float32),
            pltpu.VMEM((CS, D), jnp.float32),
            pltpu.VMEM((CS, D), jnp.float32),
            pltpu.VMEM((CS, D), jnp.float32),
            pltpu.VMEM((L,), jnp.int32),
            pltpu.SemaphoreType.DMA,
            pltpu.SemaphoreType.DMA,
            pltpu.SemaphoreType.DMA,
            pltpu.SemaphoreType.DMA,
            pltpu.SemaphoreType.DMA,
            pltpu.SemaphoreType.DMA,
        ],
        compiler_params=pltpu.CompilerParams(needs_layout_passes=False),
    )
    return run(token_ids.astype(jnp.int32), token_matrix, segment_matrix,
               pos_matrix)


# speedup vs baseline: 1.9185x; 1.0126x over previous
import jax
import jax.numpy as jnp
from jax import lax
from jax.experimental import pallas as pl
from jax.experimental.pallas import tpu as pltpu
from jax.experimental.pallas import tpu_sc as plsc

SEP = 103
D = 768
SEQ = 512
B = 32
L = 16
NC, NS = 2, 16
CS = 32
NCHUNK = SEQ // CS
DCH = D // L


def _body(ids_hbm, table_hbm, seg_hbm, pos_hbm, out_hbm,
          idx_v, t_v, seg_v, tok0, tok1, pos0, pos1, perm_v,
          sg0, sg1, sp0, sp1, so0, so1):
    c = lax.axis_index("c")
    s = lax.axis_index("s")
    w = s * NC + c

    pltpu.sync_copy(ids_hbm.at[w], idx_v)
    pltpu.sync_copy(seg_hbm, seg_v)

    iota = lax.iota(jnp.int32, L)

    def seg_body(i, carry):
        base = i * L
        tok = idx_v[pl.ds(base, L)]
        p = jnp.where(tok == SEP, 1, 0).astype(jnp.int32)
        for sh in (1, 2, 4, 8):
            perm_v[...] = p
            p = p | plsc.load_gather(perm_v, [jnp.maximum(iota - sh, 0)])
        perm_v[...] = p
        excl = plsc.load_gather(perm_v, [jnp.maximum(iota - 1, 0)])
        excl = jnp.where(iota == 0, 0, excl)
        t_v[pl.ds(base, L)] = (carry | excl).astype(jnp.float32)
        last = plsc.load_gather(perm_v, [jnp.full((L,), L - 1, jnp.int32)])
        return carry | last

    lax.fori_loop(0, SEQ // L, seg_body, jnp.zeros((L,), jnp.int32))

    def prefetch(cidx, tok_b, pos_b, sem_g, sem_p):
        base = cidx * CS
        pltpu.async_copy(table_hbm.at[idx_v.at[pl.ds(base, CS)]],
                         tok_b, sem_g)
        pltpu.async_copy(pos_hbm.at[pl.ds(base, CS)], pos_b, sem_p)

    def compute(cidx, tok_b, pos_b, sem_g, sem_p, sem_o):
        base = cidx * CS
        pltpu.make_async_copy(pos_hbm.at[pl.ds(base, CS)], pos_b,
                              sem_p).wait()
        pltpu.make_async_copy(table_hbm.at[idx_v.at[pl.ds(base, CS)]],
                              tok_b, sem_g).wait()

        tfirst = t_v[pl.ds(base, L)][0]
        tlast = t_v[pl.ds(base + CS - L, L)][L - 1]
        uniform = tfirst == tlast

        @pl.when(uniform)
        def _():
            def dloop(j, _):
                dsl = pl.ds(j * L, L)
                seg0 = seg_v[0, dsl]
                segj = seg0 + tfirst * (seg_v[1, dsl] - seg0)
                def gloop(g, _):
                    for k in range(L):
                        si = g * L + k
                        tok_b[si, dsl] = (tok_b[si, dsl] + pos_b[si, dsl]
                                          + segj)
                    return 0
                return lax.fori_loop(0, CS // L, gloop, 0)
            lax.fori_loop(0, DCH, dloop, 0)

        @pl.when(jnp.logical_not(uniform))
        def _():
            def dloop(j, _):
                dsl = pl.ds(j * L, L)
                seg0 = seg_v[0, dsl]
                dseg = seg_v[1, dsl] - seg0
                def gloop(g, _):
                    tvec = t_v[pl.ds(base + g * L, L)]
                    for k in range(L):
                        si = g * L + k
                        r = (tok_b[si, dsl] + pos_b[si, dsl]
                             + (seg0 + tvec[k] * dseg))
                        tok_b[si, dsl] = r
                    return 0
                return lax.fori_loop(0, CS // L, gloop, 0)
            lax.fori_loop(0, DCH, dloop, 0)

        pltpu.async_copy(tok_b, out_hbm.at[w, pl.ds(base, CS)], sem_o)

    def wait_out(tok_b, sem_o):
        pltpu.make_async_copy(tok_b, out_hbm.at[w, pl.ds(0, CS)],
                              sem_o).wait()

    prefetch(0, tok0, pos0, sg0, sp0)
    prefetch(1, tok1, pos1, sg1, sp1)

    def pair(h, _):
        compute(2 * h, tok0, pos0, sg0, sp0, so0)

        @pl.when(h < NCHUNK // 2 - 1)
        def _():
            wait_out(tok0, so0)
            prefetch(2 * h + 2, tok0, pos0, sg0, sp0)

        compute(2 * h + 1, tok1, pos1, sg1, sp1, so1)

        @pl.when(h < NCHUNK // 2 - 1)
        def _():
            wait_out(tok1, so1)
            prefetch(2 * h + 3, tok1, pos1, sg1, sp1)

        return 0

    lax.fori_loop(0, NCHUNK // 2, pair, 0)
    wait_out(tok0, so0)
    wait_out(tok1, so1)


@jax.jit
def kernel(token_ids, token_matrix, segment_matrix, pos_matrix):
    mesh = plsc.VectorSubcoreMesh(core_axis_name="c", subcore_axis_name="s",
                                  num_cores=NC, num_subcores=NS)
    run = pl.kernel(
        _body,
        out_type=jax.ShapeDtypeStruct((B, SEQ, D), jnp.float32),
        mesh=mesh,
        scratch_types=[
            pltpu.VMEM((SEQ,), jnp.int32),
            pltpu.VMEM((SEQ,), jnp.float32),
            pltpu.VMEM((2, D), jnp.float32),
            pltpu.VMEM((CS, D), jnp.float32),
            pltpu.VMEM((CS, D), jnp.float32),
            pltpu.VMEM((CS, D), jnp.float32),
            pltpu.VMEM((CS, D), jnp.float32),
            pltpu.VMEM((L,), jnp.int32),
            pltpu.SemaphoreType.DMA,
            pltpu.SemaphoreType.DMA,
            pltpu.SemaphoreType.DMA,
            pltpu.SemaphoreType.DMA,
            pltpu.SemaphoreType.DMA,
            pltpu.SemaphoreType.DMA,
        ],
        compiler_params=pltpu.CompilerParams(needs_layout_passes=False),
    )
    return run(token_ids.astype(jnp.int32), token_matrix, segment_matrix,
               pos_matrix)
